# Initial kernel scaffold; baseline (speedup 1.0000x reference)
import jax
import jax.numpy as jnp
from jax.experimental import pallas as pl


def kernel(boxes, scores):
    raise NotImplementedError("write your pallas kernel here")


# speedup vs baseline: 43.1721x; 43.1721x over previous
import jax
import jax.numpy as jnp
from jax import lax
from jax.experimental import pallas as pl
from jax.experimental.pallas import tpu as pltpu

_N = 5000
_NPAD = 5120
_T = 128
_NT = _NPAD // _T
_TOPN = 2000
_TH = 0.7
_IM = 512.0
_WIN = 256
_OUT_ROWS = 2304


def _iou_gt(x1c, y1c, x2c, y2c, ac, x1r, y1r, x2r, y2r, ar):
    xx1 = jnp.maximum(x1c, x1r)
    yy1 = jnp.maximum(y1c, y1r)
    xx2 = jnp.minimum(x2c, x2r)
    yy2 = jnp.minimum(y2c, y2r)
    w = jnp.clip(xx2 - xx1 + 1.0, 0.0)
    h = jnp.clip(yy2 - yy1 + 1.0, 0.0)
    inter = w * h
    iou = inter / (ac + ar - inter)
    return (iou > _TH).astype(jnp.float32)


def _load_tile(x1_ref, y1_ref, x2_ref, y2_ref, off):
    xt1 = jnp.clip(x1_ref[:, pl.ds(off, _T)], 0.0, _IM - 1.0)
    yt1 = jnp.clip(y1_ref[:, pl.ds(off, _T)], 0.0, _IM - 1.0)
    xt2 = jnp.clip(x2_ref[:, pl.ds(off, _T)], 0.0, _IM - 1.0)
    yt2 = jnp.clip(y2_ref[:, pl.ds(off, _T)], 0.0, _IM - 1.0)
    at = (xt2 - xt1 + 1.0) * (yt2 - yt1 + 1.0)
    return xt1, yt1, xt2, yt2, at


def _nms_kernel(x1_ref, y1_ref, x2_ref, y2_ref, s_ref, out_ref, alive_ref):
    pos_all = lax.broadcasted_iota(jnp.int32, (1, _NPAD), 1)
    alive_ref[...] = (pos_all < _N).astype(jnp.float32)
    out_ref[...] = jnp.zeros((_OUT_ROWS, 8), jnp.float32)

    ii = lax.broadcasted_iota(jnp.int32, (_T, _T), 0)
    jj = lax.broadcasted_iota(jnp.int32, (_T, _T), 1)
    strict_ut = (ii < jj).astype(jnp.float32)
    incl_ut = (ii <= jj).astype(jnp.float32)
    row_iota = lax.broadcasted_iota(jnp.int32, (_WIN, 1), 0).astype(
        jnp.float32)

    def colize(row):
        return jnp.broadcast_to(row, (_T, _T)).T

    def tile_body(t, base):
        off = t * _T
        xt1, yt1, xt2, yt2, at = _load_tile(x1_ref, y1_ref, x2_ref, y2_ref,
                                            off)
        st = s_ref[:, pl.ds(off, _T)]

        x1c = colize(xt1)[:, 0:1]
        y1c = colize(yt1)[:, 0:1]
        x2c = colize(xt2)[:, 0:1]
        y2c = colize(yt2)[:, 0:1]
        ac = colize(at)[:, 0:1]

        s_mat = _iou_gt(x1c, y1c, x2c, y2c, ac, xt1, yt1, xt2, yt2, at)
        s_mat = s_mat * strict_ut
        a_mask = alive_ref[:, pl.ds(off, _T)]

        def fp_cond(c):
            return c[1]

        def fp_body(c):
            k, _ = c
            sup = jax.lax.dot_general(
                k, s_mat, (((1,), (0,)), ((), ())),
                preferred_element_type=jnp.float32,
                precision=lax.Precision.HIGHEST)
            nk = jnp.where(sup > 0.0, 0.0, a_mask)
            return nk, jnp.any(nk != k)

        keep, _ = lax.while_loop(fp_cond, fp_body, (a_mask, jnp.bool_(True)))
        alive_ref[:, pl.ds(off, _T)] = keep

        def sup_body(u, _):
            offu = u * _T
            xu1, yu1, xu2, yu2, au = _load_tile(x1_ref, y1_ref, x2_ref,
                                                y2_ref, offu)
            s_u = _iou_gt(x1c, y1c, x2c, y2c, ac, xu1, yu1, xu2, yu2, au)
            supu = jax.lax.dot_general(
                keep, s_u, (((1,), (0,)), ((), ())),
                preferred_element_type=jnp.float32,
                precision=lax.Precision.HIGHEST)
            av = alive_ref[:, pl.ds(offu, _T)]
            alive_ref[:, pl.ds(offu, _T)] = jnp.where(supu > 0.0, 0.0, av)
            return 0

        lax.fori_loop(t + 1, _NT, sup_body, 0)

        pos_incl = jax.lax.dot_general(
            keep, incl_ut, (((1,), (0,)), ((), ())),
            preferred_element_type=jnp.float32,
            precision=lax.Precision.HIGHEST)
        cnt = jnp.sum(keep).astype(jnp.int32)
        posf = base.astype(jnp.float32) + pos_incl - 1.0

        base_w = jnp.minimum(base, _TOPN)
        base_al = (base_w // 8) * 8
        rel = posf - base_al.astype(jnp.float32)
        oh = ((row_iota == rel) & (keep > 0.0)
              & (posf < float(_TOPN))).astype(jnp.float32)

        cols = []
        for valr in (st, xt1, yt1, xt2, yt2):
            cols.append(jnp.sum(oh * valr, axis=1, keepdims=True))
        upd = jnp.concatenate(cols + [jnp.zeros((_WIN, 3), jnp.float32)],
                              axis=1)
        cur = out_ref[pl.ds(base_al, _WIN), :]
        out_ref[pl.ds(base_al, _WIN), :] = cur + upd
        return base + cnt

    lax.fori_loop(0, _NT, tile_body, jnp.int32(0))


def kernel(boxes, scores):
    order = jnp.argsort(-scores)
    b = boxes[order]
    s = scores[order]
    pad = _NPAD - _N
    b = jnp.pad(b, ((0, pad), (0, 0)))
    s = jnp.pad(s, ((0, pad),))
    x1 = b[:, 0][None, :]
    y1 = b[:, 1][None, :]
    x2 = b[:, 2][None, :]
    y2 = b[:, 3][None, :]
    s = s[None, :]
    out = pl.pallas_call(
        _nms_kernel,
        out_shape=jax.ShapeDtypeStruct((_OUT_ROWS, 8), jnp.float32),
        scratch_shapes=[pltpu.VMEM((1, _NPAD), jnp.float32)],
    )(x1, y1, x2, y2, s)
    return out[:_TOPN, :5]


# speedup vs baseline: 82.2433x; 1.9050x over previous
import jax
import jax.numpy as jnp
from jax import lax
from jax.experimental import pallas as pl
from jax.experimental.pallas import tpu as pltpu

_N = 5000
_NPAD = 5120
_T = 128
_NT = _NPAD // _T
_TOPN = 2000
_TH = 0.7
_IM = 512.0
_WIN = 256
_UW = 512
_OUT_ROWS = 2304


def _iou_gt(x1c, y1c, x2c, y2c, ac, x1r, y1r, x2r, y2r, ar):
    xx1 = jnp.maximum(x1c, x1r)
    yy1 = jnp.maximum(y1c, y1r)
    xx2 = jnp.minimum(x2c, x2r)
    yy2 = jnp.minimum(y2c, y2r)
    w = jnp.clip(xx2 - xx1 + 1.0, 0.0)
    h = jnp.clip(yy2 - yy1 + 1.0, 0.0)
    inter = w * h
    iou = inter / (ac + ar - inter)
    return (iou > _TH).astype(jnp.float32)


def _load_tile(x1_ref, y1_ref, x2_ref, y2_ref, off):
    xt1 = jnp.clip(x1_ref[:, pl.ds(off, _T)], 0.0, _IM - 1.0)
    yt1 = jnp.clip(y1_ref[:, pl.ds(off, _T)], 0.0, _IM - 1.0)
    xt2 = jnp.clip(x2_ref[:, pl.ds(off, _T)], 0.0, _IM - 1.0)
    yt2 = jnp.clip(y2_ref[:, pl.ds(off, _T)], 0.0, _IM - 1.0)
    at = (xt2 - xt1 + 1.0) * (yt2 - yt1 + 1.0)
    return xt1, yt1, xt2, yt2, at


def _nms_kernel(x1_ref, y1_ref, x2_ref, y2_ref, s_ref, out_ref, alive_ref):
    pos_all = lax.broadcasted_iota(jnp.int32, (1, _NPAD), 1)
    alive_ref[...] = (pos_all < _N).astype(jnp.float32)
    out_ref[...] = jnp.zeros((_OUT_ROWS, 8), jnp.float32)

    ii = lax.broadcasted_iota(jnp.int32, (_T, _T), 0)
    jj = lax.broadcasted_iota(jnp.int32, (_T, _T), 1)
    strict_ut = (ii < jj).astype(jnp.float32)
    incl_ut = (ii <= jj).astype(jnp.float32)
    row_iota = lax.broadcasted_iota(jnp.int32, (_WIN, 1), 0).astype(
        jnp.float32)
    chunk_iota = lax.broadcasted_iota(jnp.int32, (1, _UW), 1)

    def colize(row):
        return jnp.broadcast_to(row, (_T, _T)).T

    def tile_body(t, base):
        off = t * _T
        xt1, yt1, xt2, yt2, at = _load_tile(x1_ref, y1_ref, x2_ref, y2_ref,
                                            off)
        st = s_ref[:, pl.ds(off, _T)]

        x1c = colize(xt1)[:, 0:1]
        y1c = colize(yt1)[:, 0:1]
        x2c = colize(xt2)[:, 0:1]
        y2c = colize(yt2)[:, 0:1]
        ac = colize(at)[:, 0:1]

        s_mat = _iou_gt(x1c, y1c, x2c, y2c, ac, xt1, yt1, xt2, yt2, at)
        s_mat = s_mat * strict_ut
        a_mask = alive_ref[:, pl.ds(off, _T)]

        def fp_cond(c):
            return c[1]

        def fp_body(c):
            k, _ = c
            sup = jax.lax.dot_general(
                k, s_mat, (((1,), (0,)), ((), ())),
                preferred_element_type=jnp.float32,
                precision=lax.Precision.HIGHEST)
            nk = jnp.where(sup > 0.0, 0.0, a_mask)
            return nk, jnp.any(nk != k)

        keep, _ = lax.while_loop(fp_cond, fp_body, (a_mask, jnp.bool_(True)))
        alive_ref[:, pl.ds(off, _T)] = keep

        def sup_body(c, _):
            offu = c * _UW
            xu1 = jnp.clip(x1_ref[:, pl.ds(offu, _UW)], 0.0, _IM - 1.0)
            yu1 = jnp.clip(y1_ref[:, pl.ds(offu, _UW)], 0.0, _IM - 1.0)
            xu2 = jnp.clip(x2_ref[:, pl.ds(offu, _UW)], 0.0, _IM - 1.0)
            yu2 = jnp.clip(y2_ref[:, pl.ds(offu, _UW)], 0.0, _IM - 1.0)
            au = (xu2 - xu1 + 1.0) * (yu2 - yu1 + 1.0)
            s_u = _iou_gt(x1c, y1c, x2c, y2c, ac, xu1, yu1, xu2, yu2, au)
            supu = jax.lax.dot_general(
                keep, s_u, (((1,), (0,)), ((), ())),
                preferred_element_type=jnp.float32,
                precision=lax.Precision.HIGHEST)
            later = (chunk_iota + offu) > (off + _T - 1)
            av = alive_ref[:, pl.ds(offu, _UW)]
            alive_ref[:, pl.ds(offu, _UW)] = jnp.where(
                later & (supu > 0.0), 0.0, av)
            return 0

        lax.fori_loop(((t + 1) * _T) // _UW, _NPAD // _UW, sup_body, 0)

        pos_incl = jax.lax.dot_general(
            keep, incl_ut, (((1,), (0,)), ((), ())),
            preferred_element_type=jnp.float32,
            precision=lax.Precision.HIGHEST)
        cnt = jnp.sum(keep).astype(jnp.int32)
        posf = base.astype(jnp.float32) + pos_incl - 1.0

        base_w = jnp.minimum(base, _TOPN)
        base_al = (base_w // 8) * 8
        rel = posf - base_al.astype(jnp.float32)
        oh = ((row_iota == rel) & (keep > 0.0)
              & (posf < float(_TOPN))).astype(jnp.float32)

        cols = []
        for valr in (st, xt1, yt1, xt2, yt2):
            cols.append(jnp.sum(oh * valr, axis=1, keepdims=True))
        upd = jnp.concatenate(cols + [jnp.zeros((_WIN, 3), jnp.float32)],
                              axis=1)
        cur = out_ref[pl.ds(base_al, _WIN), :]
        out_ref[pl.ds(base_al, _WIN), :] = cur + upd
        return base + cnt

    def tile_step(t, base):
        return lax.cond(base < _TOPN, lambda: tile_body(t, base),
                        lambda: base)

    lax.fori_loop(0, _NT, tile_step, jnp.int32(0))


def kernel(boxes, scores):
    order = jnp.argsort(-scores)
    b = boxes[order]
    s = scores[order]
    pad = _NPAD - _N
    b = jnp.pad(b, ((0, pad), (0, 0)))
    s = jnp.pad(s, ((0, pad),))
    x1 = b[:, 0][None, :]
    y1 = b[:, 1][None, :]
    x2 = b[:, 2][None, :]
    y2 = b[:, 3][None, :]
    s = s[None, :]
    out = pl.pallas_call(
        _nms_kernel,
        out_shape=jax.ShapeDtypeStruct((_OUT_ROWS, 8), jnp.float32),
        scratch_shapes=[pltpu.VMEM((1, _NPAD), jnp.float32)],
    )(x1, y1, x2, y2, s)
    return out[:_TOPN, :5]


# speedup vs baseline: 130.1837x; 1.5829x over previous
import jax
import jax.numpy as jnp
from jax import lax
from jax.experimental import pallas as pl
from jax.experimental.pallas import tpu as pltpu

_N = 5000
_NPAD = 5120
_T = 128
_NT = _NPAD // _T
_TOPN = 2000
_TH = 0.7
_IM = 512.0
_WIN = 256
_UW = 1024
_OUT_ROWS = 2304


def _iou_gt(x1c, y1c, x2c, y2c, ac, x1r, y1r, x2r, y2r, ar):
    xx1 = jnp.maximum(x1c, x1r)
    yy1 = jnp.maximum(y1c, y1r)
    xx2 = jnp.minimum(x2c, x2r)
    yy2 = jnp.minimum(y2c, y2r)
    w = jnp.clip(xx2 - xx1 + 1.0, 0.0)
    h = jnp.clip(yy2 - yy1 + 1.0, 0.0)
    inter = w * h
    iou = inter / (ac + ar - inter)
    return (iou > _TH).astype(jnp.float32)


def _load_tile(x1_ref, y1_ref, x2_ref, y2_ref, off):
    xt1 = jnp.clip(x1_ref[:, pl.ds(off, _T)], 0.0, _IM - 1.0)
    yt1 = jnp.clip(y1_ref[:, pl.ds(off, _T)], 0.0, _IM - 1.0)
    xt2 = jnp.clip(x2_ref[:, pl.ds(off, _T)], 0.0, _IM - 1.0)
    yt2 = jnp.clip(y2_ref[:, pl.ds(off, _T)], 0.0, _IM - 1.0)
    at = (xt2 - xt1 + 1.0) * (yt2 - yt1 + 1.0)
    return xt1, yt1, xt2, yt2, at


def _nms_kernel(x1_ref, y1_ref, x2_ref, y2_ref, s_ref, out_ref, alive_ref):
    pos_all = lax.broadcasted_iota(jnp.int32, (1, _NPAD), 1)
    alive_ref[...] = (pos_all < _N).astype(jnp.float32)
    out_ref[...] = jnp.zeros((_OUT_ROWS, 8), jnp.float32)

    ii = lax.broadcasted_iota(jnp.int32, (_T, _T), 0)
    jj = lax.broadcasted_iota(jnp.int32, (_T, _T), 1)
    strict_ut = (ii < jj).astype(jnp.float32)
    incl_ut = (ii <= jj).astype(jnp.float32)
    row_iota = lax.broadcasted_iota(jnp.int32, (_WIN, 1), 0).astype(
        jnp.float32)
    chunk_iota = lax.broadcasted_iota(jnp.int32, (1, _UW), 1)

    def colize(row):
        return jnp.broadcast_to(row, (_T, _T)).T

    def tile_body(t, base):
        off = t * _T
        xt1, yt1, xt2, yt2, at = _load_tile(x1_ref, y1_ref, x2_ref, y2_ref,
                                            off)
        st = s_ref[:, pl.ds(off, _T)]

        x1c = colize(xt1)[:, 0:1]
        y1c = colize(yt1)[:, 0:1]
        x2c = colize(xt2)[:, 0:1]
        y2c = colize(yt2)[:, 0:1]
        ac = colize(at)[:, 0:1]

        s_mat = _iou_gt(x1c, y1c, x2c, y2c, ac, xt1, yt1, xt2, yt2, at)
        s_mat = s_mat * strict_ut
        a_mask = alive_ref[:, pl.ds(off, _T)]

        def fp_cond(c):
            return c[1]

        def fp_body(c):
            k, _ = c
            sup = jax.lax.dot_general(
                k, s_mat, (((1,), (0,)), ((), ())),
                preferred_element_type=jnp.float32,
                precision=lax.Precision.HIGHEST)
            nk = jnp.where(sup > 0.0, 0.0, a_mask)
            return nk, jnp.any(nk != k)

        keep, _ = lax.while_loop(fp_cond, fp_body, (a_mask, jnp.bool_(True)))
        alive_ref[:, pl.ds(off, _T)] = keep

        def sup_body(c, _):
            offu = c * _UW
            xu1 = jnp.clip(x1_ref[:, pl.ds(offu, _UW)], 0.0, _IM - 1.0)
            yu1 = jnp.clip(y1_ref[:, pl.ds(offu, _UW)], 0.0, _IM - 1.0)
            xu2 = jnp.clip(x2_ref[:, pl.ds(offu, _UW)], 0.0, _IM - 1.0)
            yu2 = jnp.clip(y2_ref[:, pl.ds(offu, _UW)], 0.0, _IM - 1.0)
            au = (xu2 - xu1 + 1.0) * (yu2 - yu1 + 1.0)
            s_u = _iou_gt(x1c, y1c, x2c, y2c, ac, xu1, yu1, xu2, yu2, au)
            supu = jax.lax.dot_general(
                keep, s_u, (((1,), (0,)), ((), ())),
                preferred_element_type=jnp.float32,
                precision=lax.Precision.HIGHEST)
            later = (chunk_iota + offu) > (off + _T - 1)
            av = alive_ref[:, pl.ds(offu, _UW)]
            alive_ref[:, pl.ds(offu, _UW)] = jnp.where(
                later & (supu > 0.0), 0.0, av)
            return 0

        lax.fori_loop(((t + 1) * _T) // _UW, _NPAD // _UW, sup_body, 0)

        pos_incl = jax.lax.dot_general(
            keep, incl_ut, (((1,), (0,)), ((), ())),
            preferred_element_type=jnp.float32,
            precision=lax.Precision.HIGHEST)
        cnt = jnp.sum(keep).astype(jnp.int32)
        posf = base.astype(jnp.float32) + pos_incl - 1.0

        base_w = jnp.minimum(base, _TOPN)
        base_al = (base_w // 8) * 8
        rel = posf - base_al.astype(jnp.float32)
        oh = ((row_iota == rel) & (keep > 0.0)
              & (posf < float(_TOPN))).astype(jnp.float32)

        cols = []
        for valr in (st, xt1, yt1, xt2, yt2):
            cols.append(jnp.sum(oh * valr, axis=1, keepdims=True))
        upd = jnp.concatenate(cols + [jnp.zeros((_WIN, 3), jnp.float32)],
                              axis=1)
        cur = out_ref[pl.ds(base_al, _WIN), :]
        out_ref[pl.ds(base_al, _WIN), :] = cur + upd
        return base + cnt

    def tile_step(t, base):
        return lax.cond(base < _TOPN, lambda: tile_body(t, base),
                        lambda: base)

    lax.fori_loop(0, _NT, tile_step, jnp.int32(0))


def kernel(boxes, scores):
    _, x1, y1, x2, y2, s = jax.lax.sort(
        (-scores, boxes[:, 0], boxes[:, 1], boxes[:, 2], boxes[:, 3],
         scores),
        num_keys=1, is_stable=True)
    pad = _NPAD - _N
    x1 = jnp.pad(x1, ((0, pad),))[None, :]
    y1 = jnp.pad(y1, ((0, pad),))[None, :]
    x2 = jnp.pad(x2, ((0, pad),))[None, :]
    y2 = jnp.pad(y2, ((0, pad),))[None, :]
    s = jnp.pad(s, ((0, pad),))[None, :]
    out = pl.pallas_call(
        _nms_kernel,
        out_shape=jax.ShapeDtypeStruct((_OUT_ROWS, 8), jnp.float32),
        scratch_shapes=[pltpu.VMEM((1, _NPAD), jnp.float32)],
    )(x1, y1, x2, y2, s)
    return out[:_TOPN, :5]


# speedup vs baseline: 162.5613x; 1.2487x over previous
import jax
import jax.numpy as jnp
from jax import lax
from jax.experimental import pallas as pl
from jax.experimental.pallas import tpu as pltpu

_N = 5000
_NPAD = 5120
_T = 128
_NT = _NPAD // _T
_TOPN = 2000
_TH = 0.7
_IM = 512.0
_WIN = 256
_PW = 1024
_OUT_ROWS = 2304


def _iou_gt(x1c, y1c, x2c, y2c, ac, x1r, y1r, x2r, y2r, ar):
    xx1 = jnp.maximum(x1c, x1r)
    yy1 = jnp.maximum(y1c, y1r)
    xx2 = jnp.minimum(x2c, x2r)
    yy2 = jnp.minimum(y2c, y2r)
    w = jnp.clip(xx2 - xx1 + 1.0, 0.0)
    h = jnp.clip(yy2 - yy1 + 1.0, 0.0)
    inter = w * h
    iou = inter / (ac + ar - inter)
    return (iou > _TH).astype(jnp.float32)


def _load_tile(x1_ref, y1_ref, x2_ref, y2_ref, off):
    xt1 = jnp.clip(x1_ref[:, pl.ds(off, _T)], 0.0, _IM - 1.0)
    yt1 = jnp.clip(y1_ref[:, pl.ds(off, _T)], 0.0, _IM - 1.0)
    xt2 = jnp.clip(x2_ref[:, pl.ds(off, _T)], 0.0, _IM - 1.0)
    yt2 = jnp.clip(y2_ref[:, pl.ds(off, _T)], 0.0, _IM - 1.0)
    at = (xt2 - xt1 + 1.0) * (yt2 - yt1 + 1.0)
    return xt1, yt1, xt2, yt2, at


def _nms_kernel(x1_ref, y1_ref, x2_ref, y2_ref, s_ref, out_ref, keep_ref,
                cols_ref):
    keep_ref[...] = jnp.zeros((1, _NPAD), jnp.float32)
    cols_ref[...] = jnp.zeros((_NPAD, 8), jnp.float32)
    out_ref[...] = jnp.zeros((_OUT_ROWS, 8), jnp.float32)

    ii = lax.broadcasted_iota(jnp.int32, (_T, _T), 0)
    jj = lax.broadcasted_iota(jnp.int32, (_T, _T), 1)
    strict_ut = (ii < jj).astype(jnp.float32)
    incl_ut = (ii <= jj).astype(jnp.float32)
    row_iota = lax.broadcasted_iota(jnp.int32, (_WIN, 1), 0).astype(
        jnp.float32)
    lane_iota = lax.broadcasted_iota(jnp.int32, (1, _T), 1)

    def colize(row):
        return jnp.broadcast_to(row, (_T, _T)).T

    def tile_body(t, base):
        off = t * _T
        xt1, yt1, xt2, yt2, at = _load_tile(x1_ref, y1_ref, x2_ref, y2_ref,
                                            off)
        st = s_ref[:, pl.ds(off, _T)]

        x1c = colize(xt1)[:, 0:1]
        y1c = colize(yt1)[:, 0:1]
        x2c = colize(xt2)[:, 0:1]
        y2c = colize(yt2)[:, 0:1]
        ac = colize(at)[:, 0:1]

        def pre_body(c, acc):
            poff = c * _PW
            px1 = cols_ref[pl.ds(poff, _PW), 0:1]
            py1 = cols_ref[pl.ds(poff, _PW), 1:2]
            px2 = cols_ref[pl.ds(poff, _PW), 2:3]
            py2 = cols_ref[pl.ds(poff, _PW), 3:4]
            pa = cols_ref[pl.ds(poff, _PW), 4:5]
            s_c = _iou_gt(px1, py1, px2, py2, pa, xt1, yt1, xt2, yt2, at)
            kp = keep_ref[:, pl.ds(poff, _PW)]
            return acc + jax.lax.dot_general(
                kp, s_c, (((1,), (0,)), ((), ())),
                preferred_element_type=jnp.float32,
                precision=lax.Precision.HIGHEST)

        sup = lax.fori_loop(0, (t * _T + _PW - 1) // _PW, pre_body,
                            jnp.zeros((1, _T), jnp.float32))
        a_mask = jnp.where(
            (sup > 0.0) | ((lane_iota + off) >= _N), 0.0, 1.0)

        s_mat = _iou_gt(x1c, y1c, x2c, y2c, ac, xt1, yt1, xt2, yt2, at)
        s_mat = s_mat * strict_ut

        def fp_cond(c):
            return c[1]

        def fp_body(c):
            k, _ = c
            sup = jax.lax.dot_general(
                k, s_mat, (((1,), (0,)), ((), ())),
                preferred_element_type=jnp.float32,
                precision=lax.Precision.HIGHEST)
            nk = jnp.where(sup > 0.0, 0.0, a_mask)
            return nk, jnp.any(nk != k)

        keep, _ = lax.while_loop(fp_cond, fp_body, (a_mask, jnp.bool_(True)))
        keep_ref[:, pl.ds(off, _T)] = keep
        cols_ref[pl.ds(off, _T), :] = jnp.concatenate(
            [x1c, y1c, x2c, y2c, ac, jnp.zeros((_T, 3), jnp.float32)],
            axis=1)

        pos_incl = jax.lax.dot_general(
            keep, incl_ut, (((1,), (0,)), ((), ())),
            preferred_element_type=jnp.float32,
            precision=lax.Precision.HIGHEST)
        cnt = jnp.sum(keep).astype(jnp.int32)
        posf = base.astype(jnp.float32) + pos_incl - 1.0

        base_w = jnp.minimum(base, _TOPN)
        base_al = (base_w // 8) * 8
        rel = posf - base_al.astype(jnp.float32)
        oh = ((row_iota == rel) & (keep > 0.0)
              & (posf < float(_TOPN))).astype(jnp.float32)

        cols = []
        for valr in (st, xt1, yt1, xt2, yt2):
            cols.append(jnp.sum(oh * valr, axis=1, keepdims=True))
        upd = jnp.concatenate(cols + [jnp.zeros((_WIN, 3), jnp.float32)],
                              axis=1)
        cur = out_ref[pl.ds(base_al, _WIN), :]
        out_ref[pl.ds(base_al, _WIN), :] = cur + upd
        return base + cnt

    def tile_step(t, base):
        return lax.cond(base < _TOPN, lambda: tile_body(t, base),
                        lambda: base)

    lax.fori_loop(0, _NT, tile_step, jnp.int32(0))


def kernel(boxes, scores):
    _, x1, y1, x2, y2, s = jax.lax.sort(
        (-scores, boxes[:, 0], boxes[:, 1], boxes[:, 2], boxes[:, 3],
         scores),
        num_keys=1, is_stable=True)
    pad = _NPAD - _N
    x1 = jnp.pad(x1, ((0, pad),))[None, :]
    y1 = jnp.pad(y1, ((0, pad),))[None, :]
    x2 = jnp.pad(x2, ((0, pad),))[None, :]
    y2 = jnp.pad(y2, ((0, pad),))[None, :]
    s = jnp.pad(s, ((0, pad),))[None, :]
    out = pl.pallas_call(
        _nms_kernel,
        out_shape=jax.ShapeDtypeStruct((_OUT_ROWS, 8), jnp.float32),
        scratch_shapes=[pltpu.VMEM((1, _NPAD), jnp.float32),
                        pltpu.VMEM((_NPAD, 8), jnp.float32)],
    )(x1, y1, x2, y2, s)
    return out[:_TOPN, :5]


# speedup vs baseline: 164.9559x; 1.0147x over previous
import jax
import jax.numpy as jnp
from jax import lax
from jax.experimental import pallas as pl

_N = 5000
_NPAD = 5120
_T = 128
_NT = _NPAD // _T
_TOPN = 2000
_TH = 0.7
_IM = 512.0
_WIN = 256
_PW = 1024
_OUT_ROWS = 2304

_DN = (((1,), (0,)), ((), ()))
_HI = lax.Precision.HIGHEST


def _iou_gt(x1c, y1c, x2c, y2c, ac, x1r, y1r, x2r, y2r, ar):
    xx1 = jnp.maximum(x1c, x1r)
    yy1 = jnp.maximum(y1c, y1r)
    xx2 = jnp.minimum(x2c, x2r)
    yy2 = jnp.minimum(y2c, y2r)
    w = jnp.clip(xx2 - xx1 + 1.0, 0.0)
    h = jnp.clip(yy2 - yy1 + 1.0, 0.0)
    inter = w * h
    iou = inter / (ac + ar - inter)
    return (iou > _TH).astype(jnp.float32)


def _nms_kernel(x1_ref, y1_ref, x2_ref, y2_ref, s_ref, out_ref):
    out_ref[...] = jnp.zeros((_OUT_ROWS, 8), jnp.float32)

    ii = lax.broadcasted_iota(jnp.int32, (_T, _T), 0)
    jj = lax.broadcasted_iota(jnp.int32, (_T, _T), 1)
    strict_ut = (ii < jj).astype(jnp.float32)
    incl_ut = (ii <= jj).astype(jnp.float32)
    ident = (ii == jj).astype(jnp.float32)
    row_iota = lax.broadcasted_iota(jnp.int32, (_WIN, 1), 0).astype(
        jnp.float32)
    lane_iota = lax.broadcasted_iota(jnp.int32, (1, _T), 1)
    ones_pw = jnp.ones((1, _PW), jnp.float32)

    def tile_body(t, base):
        off = t * _T
        xt1 = jnp.clip(x1_ref[:, pl.ds(off, _T)], 0.0, _IM - 1.0)
        yt1 = jnp.clip(y1_ref[:, pl.ds(off, _T)], 0.0, _IM - 1.0)
        xt2 = jnp.clip(x2_ref[:, pl.ds(off, _T)], 0.0, _IM - 1.0)
        yt2 = jnp.clip(y2_ref[:, pl.ds(off, _T)], 0.0, _IM - 1.0)
        at = (xt2 - xt1 + 1.0) * (yt2 - yt1 + 1.0)
        st = s_ref[:, pl.ds(off, _T)]

        stacked8 = jnp.concatenate(
            [st, xt1, yt1, xt2, yt2, at, jnp.zeros((2, _T), jnp.float32)],
            axis=0)
        cols8 = lax.dot_general(ident, stacked8, (((1,), (1,)), ((), ())),
                                preferred_element_type=jnp.float32,
                                precision=_HI)
        x1c = cols8[:, 1:2]
        y1c = cols8[:, 2:3]
        x2c = cols8[:, 3:4]
        y2c = cols8[:, 4:5]
        ac = cols8[:, 5:6]

        def pre_body(c, acc):
            poff = c * _PW
            px1 = out_ref[pl.ds(poff, _PW), 1:2]
            py1 = out_ref[pl.ds(poff, _PW), 2:3]
            px2 = out_ref[pl.ds(poff, _PW), 3:4]
            py2 = out_ref[pl.ds(poff, _PW), 4:5]
            pa = out_ref[pl.ds(poff, _PW), 5:6]
            s_c = _iou_gt(px1, py1, px2, py2, pa, xt1, yt1, xt2, yt2, at)
            return acc + lax.dot_general(
                ones_pw, s_c, _DN, preferred_element_type=jnp.float32,
                precision=_HI)

        sup = lax.fori_loop(0, (base + _PW - 1) // _PW, pre_body,
                            jnp.zeros((1, _T), jnp.float32))
        a_mask = jnp.where(
            (sup > 0.0) | ((lane_iota + off) >= _N), 0.0, 1.0)

        s_mat = _iou_gt(x1c, y1c, x2c, y2c, ac, xt1, yt1, xt2, yt2, at)
        s_mat = s_mat * strict_ut

        def fp_cond(c):
            return c[1]

        def fp_body(c):
            k, _ = c
            sup_k = lax.dot_general(k, s_mat, _DN,
                                    preferred_element_type=jnp.float32,
                                    precision=_HI)
            nk = jnp.where(sup_k > 0.0, 0.0, a_mask)
            return nk, jnp.any(nk != k)

        keep, _ = lax.while_loop(fp_cond, fp_body, (a_mask, jnp.bool_(True)))

        pos_incl = lax.dot_general(keep, incl_ut, _DN,
                                   preferred_element_type=jnp.float32,
                                   precision=_HI)
        cnt = jnp.sum(keep).astype(jnp.int32)
        posf = base.astype(jnp.float32) + pos_incl - 1.0

        base_al = (jnp.minimum(base, _TOPN) // 8) * 8
        rel = posf - base_al.astype(jnp.float32)
        oh = ((row_iota == rel) & (keep > 0.0)
              & (posf < float(_TOPN))).astype(jnp.float32)
        upd = lax.dot_general(oh, cols8, _DN,
                              preferred_element_type=jnp.float32,
                              precision=_HI)
        cur = out_ref[pl.ds(base_al, _WIN), :]
        out_ref[pl.ds(base_al, _WIN), :] = cur + upd
        return base + cnt

    def tile_step(t, base):
        return lax.cond(base < _TOPN, lambda: tile_body(t, base),
                        lambda: base)

    lax.fori_loop(0, _NT, tile_step, jnp.int32(0))


def kernel(boxes, scores):
    _, x1, y1, x2, y2, s = jax.lax.sort(
        (-scores, boxes[:, 0], boxes[:, 1], boxes[:, 2], boxes[:, 3],
         scores),
        num_keys=1, is_stable=True)
    pad = _NPAD - _N
    x1 = jnp.pad(x1, ((0, pad),))[None, :]
    y1 = jnp.pad(y1, ((0, pad),))[None, :]
    x2 = jnp.pad(x2, ((0, pad),))[None, :]
    y2 = jnp.pad(y2, ((0, pad),))[None, :]
    s = jnp.pad(s, ((0, pad),))[None, :]
    out = pl.pallas_call(
        _nms_kernel,
        out_shape=jax.ShapeDtypeStruct((_OUT_ROWS, 8), jnp.float32),
    )(x1, y1, x2, y2, s)
    return out[:_TOPN, :5]
